# Initial kernel scaffold; baseline (speedup 1.0000x reference)
import jax
import jax.numpy as jnp
from jax.experimental import pallas as pl


def kernel(x, adj, Ws, As, W_out, a_out):
    raise NotImplementedError("write your pallas kernel here")


# speedup vs baseline: 2.4870x; 2.4870x over previous
import jax
import jax.numpy as jnp
from jax.experimental import pallas as pl

_LEAK = 0.2
_MASKVAL = -9e15


def _proj_body(x_ref, w_ref, asrc_ref, adst_ref, h_ref, ls_ref, ld_ref):
    h = jnp.dot(x_ref[...], w_ref[...], preferred_element_type=jnp.float32)
    h_ref[...] = h
    ls_ref[...] = jnp.dot(h, asrc_ref[...], preferred_element_type=jnp.float32)
    ld_ref[...] = jnp.dot(h, adst_ref[...], preferred_element_type=jnp.float32)


def _project(x, w, asrc, adst):
    n, _ = x.shape
    f = w.shape[1]
    h, ls, ld = pl.pallas_call(
        _proj_body,
        out_shape=[
            jax.ShapeDtypeStruct((n, f), jnp.float32),
            jax.ShapeDtypeStruct((n, 8), jnp.float32),
            jax.ShapeDtypeStruct((n, 8), jnp.float32),
        ],
    )(x, w, asrc, adst)
    return h, ls, ld


def _attn_body(n_mechs, f, adj_ref, ls_ref, ldt_ref, h_ref, out_ref):
    mask = adj_ref[...] > 0.0
    for m in range(n_mechs):
        s = ls_ref[:, m : m + 1] + ldt_ref[m : m + 1, :]
        e = jnp.maximum(s, _LEAK * s)
        e = jnp.where(mask, e, _MASKVAL)
        mx = jnp.max(e, axis=1, keepdims=True)
        p = jnp.exp(e - mx)
        den = jnp.sum(p, axis=1, keepdims=True)
        acc = jnp.dot(p, h_ref[:, m * f : (m + 1) * f],
                      preferred_element_type=jnp.float32)
        acc = acc / den
        out_ref[:, m * f : (m + 1) * f] = jnp.where(
            acc > 0.0, acc, jnp.exp(acc) - 1.0)


def _attention(adj, ls, ldt, h, n_mechs, f, block_rows):
    n = adj.shape[0]
    body = lambda a, b, c, d, o: _attn_body(n_mechs, f, a, b, c, d, o)
    return pl.pallas_call(
        body,
        grid=(n // block_rows,),
        in_specs=[
            pl.BlockSpec((block_rows, n), lambda i: (i, 0)),
            pl.BlockSpec((block_rows, 8), lambda i: (i, 0)),
            pl.BlockSpec((8, n), lambda i: (0, 0)),
            pl.BlockSpec((n, n_mechs * f), lambda i: (0, 0)),
        ],
        out_specs=pl.BlockSpec((block_rows, n_mechs * f), lambda i: (i, 0)),
        out_shape=jax.ShapeDtypeStruct((n, n_mechs * f), jnp.float32),
    )(adj, ls, ldt, h)


def kernel(x, adj, Ws, As, W_out, a_out):
    n, ins = x.shape
    n_mechs, _, f1 = Ws.shape
    out_dim = W_out.shape[1]
    block_rows = 200 if n % 200 == 0 else (8 if n % 8 == 0 else 1)

    w_cat = jnp.transpose(Ws, (1, 0, 2)).reshape(ins, n_mechs * f1)
    asrc = jnp.zeros((n_mechs * f1, 8), jnp.float32)
    adst = jnp.zeros((n_mechs * f1, 8), jnp.float32)
    for m in range(n_mechs):
        asrc = asrc.at[m * f1 : (m + 1) * f1, m].set(As[m, :f1])
        adst = adst.at[m * f1 : (m + 1) * f1, m].set(As[m, f1:])

    h1, ls1, ld1 = _project(x, w_cat, asrc, adst)
    xc = _attention(adj, ls1, ld1.T, h1, n_mechs, f1, block_rows)

    a2s = jnp.zeros((out_dim, 8), jnp.float32).at[:, 0].set(a_out[:out_dim])
    a2d = jnp.zeros((out_dim, 8), jnp.float32).at[:, 0].set(a_out[out_dim:])
    h2, ls2, ld2 = _project(xc, W_out, a2s, a2d)
    out = _attention(adj, ls2, ld2.T, h2, 1, out_dim, block_rows)
    return out


# speedup vs baseline: 2.7947x; 1.1237x over previous
import jax
import jax.numpy as jnp
from jax.experimental import pallas as pl

_LEAK = 0.2
_MASKVAL = -9e15


def _proj_body(f, n_mechs, x_ref, w_ref, asrc_ref, adst_ref,
               h_ref, ls_ref, ld_ref):
    h = jnp.dot(x_ref[...], w_ref[...], preferred_element_type=jnp.float32)
    ls_ref[...] = jnp.dot(h, asrc_ref[...],
                          preferred_element_type=jnp.float32
                          ).astype(jnp.bfloat16)
    ld_ref[...] = jnp.dot(h, adst_ref[...],
                          preferred_element_type=jnp.float32
                          ).astype(jnp.bfloat16)
    stride = f + 8
    hb = h.astype(jnp.bfloat16)
    for m in range(n_mechs):
        h_ref[:, m * stride : m * stride + f] = hb[:, m * f : (m + 1) * f]
        h_ref[:, m * stride + f : m * stride + stride] = jnp.ones(
            (h.shape[0], 8), jnp.bfloat16)


def _project(x, w, asrc, adst, n_mechs, f):
    n = x.shape[0]
    body = lambda *refs: _proj_body(f, n_mechs, *refs)
    h, ls, ld = pl.pallas_call(
        body,
        out_shape=[
            jax.ShapeDtypeStruct((n, n_mechs * (f + 8)), jnp.bfloat16),
            jax.ShapeDtypeStruct((n, 8), jnp.bfloat16),
            jax.ShapeDtypeStruct((n, 8), jnp.bfloat16),
        ],
    )(x, w, asrc, adst)
    return h, ls, ld


def _attn_body(n_mechs, f, adj_ref, ls_ref, ldt_ref, h_ref, out_ref):
    adjb = adj_ref[...].astype(jnp.bfloat16)
    madd = (adjb - 1.0) * jnp.bfloat16(-_MASKVAL)
    stride = f + 8
    for m in range(n_mechs):
        s = ls_ref[:, m : m + 1] + ldt_ref[m : m + 1, :]
        e = jnp.maximum(s, jnp.bfloat16(_LEAK) * s)
        e = e * adjb + madd
        mx = jnp.max(e, axis=1, keepdims=True)
        p = jnp.exp(e - mx)
        acc = jnp.dot(p, h_ref[:, m * stride : (m + 1) * stride],
                      preferred_element_type=jnp.float32)
        o = acc[:, :f] / acc[:, f : f + 1]
        out_ref[:, m * f : (m + 1) * f] = jnp.where(
            o > 0.0, o, jnp.exp(o) - 1.0)


def _attention(adj, ls, ldt, h, n_mechs, f, block_rows):
    n = adj.shape[0]
    body = lambda a, b, c, d, o: _attn_body(n_mechs, f, a, b, c, d, o)
    return pl.pallas_call(
        body,
        grid=(n // block_rows,),
        in_specs=[
            pl.BlockSpec((block_rows, n), lambda i: (i, 0)),
            pl.BlockSpec((block_rows, 8), lambda i: (i, 0)),
            pl.BlockSpec((8, n), lambda i: (0, 0)),
            pl.BlockSpec((n, n_mechs * (f + 8)), lambda i: (0, 0)),
        ],
        out_specs=pl.BlockSpec((block_rows, n_mechs * f), lambda i: (i, 0)),
        out_shape=jax.ShapeDtypeStruct((n, n_mechs * f), jnp.float32),
    )(adj, ls, ldt, h)


def kernel(x, adj, Ws, As, W_out, a_out):
    n, ins = x.shape
    n_mechs, _, f1 = Ws.shape
    out_dim = W_out.shape[1]
    block_rows = 200 if n % 200 == 0 else (8 if n % 8 == 0 else 1)

    w_cat = jnp.transpose(Ws, (1, 0, 2)).reshape(ins, n_mechs * f1)
    asrc = jnp.zeros((n_mechs * f1, 8), jnp.float32)
    adst = jnp.zeros((n_mechs * f1, 8), jnp.float32)
    for m in range(n_mechs):
        asrc = asrc.at[m * f1 : (m + 1) * f1, m].set(As[m, :f1])
        adst = adst.at[m * f1 : (m + 1) * f1, m].set(As[m, f1:])

    h1, ls1, ld1 = _project(x, w_cat, asrc, adst, n_mechs, f1)
    xc = _attention(adj, ls1, ld1.T, h1, n_mechs, f1, block_rows)

    a2s = jnp.zeros((out_dim, 8), jnp.float32).at[:, 0].set(a_out[:out_dim])
    a2d = jnp.zeros((out_dim, 8), jnp.float32).at[:, 0].set(a_out[out_dim:])
    h2, ls2, ld2 = _project(xc, W_out, a2s, a2d, 1, out_dim)
    out = _attention(adj, ls2, ld2.T, h2, 1, out_dim, block_rows)
    return out


# speedup vs baseline: 5.0255x; 1.7982x over previous
import jax
import jax.numpy as jnp
from jax.experimental import pallas as pl

_LEAK = 0.2
_LOG2E = 1.4426950408889634
_NEGM = -9e15 * _LOG2E
_BIG = 1e30


def _proj_body(f, n_mechs, x_ref, w_ref, asrc_ref, adst_ref,
               h_ref, ls_ref, ld_ref, mp_ref, hmean_ref):
    h = jnp.dot(x_ref[...], w_ref[...], preferred_element_type=jnp.float32)
    ls = jnp.dot(h, asrc_ref[...], preferred_element_type=jnp.float32)
    ld = jnp.dot(h, adst_ref[...], preferred_element_type=jnp.float32)
    ls_ref[...] = ls.astype(jnp.bfloat16)
    ld_ref[...] = ld.astype(jnp.bfloat16)
    b = ls + jnp.max(ld, axis=0, keepdims=True)
    mp_ref[...] = jnp.maximum(b, _LEAK * b).astype(jnp.bfloat16)
    hmean_ref[...] = jnp.broadcast_to(
        jnp.mean(h, axis=0, keepdims=True), hmean_ref.shape)
    stride = f + 8
    hb = h.astype(jnp.bfloat16)
    for m in range(n_mechs):
        h_ref[:, m * stride : m * stride + f] = hb[:, m * f : (m + 1) * f]
        h_ref[:, m * stride + f : m * stride + stride] = jnp.ones(
            (h.shape[0], 8), jnp.bfloat16)


def _project(x, w, asrc, adst, n_mechs, f):
    n = x.shape[0]
    body = lambda *refs: _proj_body(f, n_mechs, *refs)
    return pl.pallas_call(
        body,
        out_shape=[
            jax.ShapeDtypeStruct((n, n_mechs * (f + 8)), jnp.bfloat16),
            jax.ShapeDtypeStruct((n, 8), jnp.bfloat16),
            jax.ShapeDtypeStruct((n, 8), jnp.bfloat16),
            jax.ShapeDtypeStruct((n, 8), jnp.bfloat16),
            jax.ShapeDtypeStruct((8, n_mechs * f), jnp.float32),
        ],
    )(x, w, asrc, adst)


def _mech_loop(n_mechs, f, bigm, ls_ref, ldt_ref, mp_ref, h_ref,
               hmean_ref, out_ref):
    stride = f + 8
    for m in range(n_mechs):
        s = ls_ref[:, m : m + 1] + ldt_ref[m : m + 1, :]
        e = jnp.maximum(s, jnp.bfloat16(_LEAK) * s)
        t = jnp.minimum(e, bigm) - mp_ref[:, m : m + 1]
        p = jnp.exp2(t)
        acc = jnp.dot(p, h_ref[:, m * stride : (m + 1) * stride],
                      preferred_element_type=jnp.float32)
        den = acc[:, f : f + 1]
        o = jnp.where(den > 0.0, acc[:, :f] / den,
                      hmean_ref[0:1, m * f : (m + 1) * f])
        out_ref[:, m * f : (m + 1) * f] = jnp.where(
            o > 0.0, o, jnp.exp(o) - 1.0)


def _attn1_body(n_mechs, f, adj_ref, ls_ref, ldt_ref, mp_ref, h_ref,
                hmean_ref, out_ref, bigm_ref):
    bigm = (adj_ref[...] * _BIG + _NEGM).astype(jnp.bfloat16)
    bigm_ref[...] = bigm
    _mech_loop(n_mechs, f, bigm, ls_ref, ldt_ref, mp_ref, h_ref,
               hmean_ref, out_ref)


def _attn2_body(n_mechs, f, bigm_ref, ls_ref, ldt_ref, mp_ref, h_ref,
                hmean_ref, out_ref):
    _mech_loop(n_mechs, f, bigm_ref[...], ls_ref, ldt_ref, mp_ref, h_ref,
               hmean_ref, out_ref)


def _attention(adj, ls, ldt, mp, h, hmean, n_mechs, f, block_rows,
               emit_bigm):
    n = adj.shape[0]
    row_specs = [
        pl.BlockSpec((block_rows, n), lambda i: (i, 0)),
        pl.BlockSpec((block_rows, 8), lambda i: (i, 0)),
        pl.BlockSpec((8, n), lambda i: (0, 0)),
        pl.BlockSpec((block_rows, 8), lambda i: (i, 0)),
        pl.BlockSpec((n, n_mechs * (f + 8)), lambda i: (0, 0)),
        pl.BlockSpec((8, n_mechs * f), lambda i: (0, 0)),
    ]
    out_spec = pl.BlockSpec((block_rows, n_mechs * f), lambda i: (i, 0))
    if emit_bigm:
        body = lambda *refs: _attn1_body(n_mechs, f, *refs)
        return pl.pallas_call(
            body,
            grid=(n // block_rows,),
            in_specs=row_specs,
            out_specs=[out_spec,
                       pl.BlockSpec((block_rows, n), lambda i: (i, 0))],
            out_shape=[
                jax.ShapeDtypeStruct((n, n_mechs * f), jnp.float32),
                jax.ShapeDtypeStruct((n, n), jnp.bfloat16),
            ],
        )(adj, ls, ldt, mp, h, hmean)
    body = lambda *refs: _attn2_body(n_mechs, f, *refs)
    return pl.pallas_call(
        body,
        grid=(n // block_rows,),
        in_specs=row_specs,
        out_specs=out_spec,
        out_shape=jax.ShapeDtypeStruct((n, n_mechs * f), jnp.float32),
    )(adj, ls, ldt, mp, h, hmean)


def kernel(x, adj, Ws, As, W_out, a_out):
    n, ins = x.shape
    n_mechs, _, f1 = Ws.shape
    out_dim = W_out.shape[1]
    block_rows = 200 if n % 200 == 0 else (8 if n % 8 == 0 else 1)

    w_cat = jnp.transpose(Ws, (1, 0, 2)).reshape(ins, n_mechs * f1)
    asrc = jnp.zeros((n_mechs * f1, 8), jnp.float32)
    adst = jnp.zeros((n_mechs * f1, 8), jnp.float32)
    for m in range(n_mechs):
        asrc = asrc.at[m * f1 : (m + 1) * f1, m].set(As[m, :f1] * _LOG2E)
        adst = adst.at[m * f1 : (m + 1) * f1, m].set(As[m, f1:] * _LOG2E)

    h1, ls1, ld1, mp1, hm1 = _project(x, w_cat, asrc, adst, n_mechs, f1)
    xc, bigm = _attention(adj, ls1, ld1.T, mp1, h1, hm1, n_mechs, f1,
                          block_rows, True)

    a2s = jnp.zeros((out_dim, 8), jnp.float32).at[:, 0].set(
        a_out[:out_dim] * _LOG2E)
    a2d = jnp.zeros((out_dim, 8), jnp.float32).at[:, 0].set(
        a_out[out_dim:] * _LOG2E)
    h2, ls2, ld2, mp2, hm2 = _project(xc, W_out, a2s, a2d, 1, out_dim)
    out = _attention(bigm, ls2, ld2.T, mp2, h2, hm2, 1, out_dim,
                     block_rows, False)
    return out


# speedup vs baseline: 5.4112x; 1.0768x over previous
import jax
import jax.numpy as jnp
from jax.experimental import pallas as pl

_LEAK = 0.2
_LOG2E = 1.4426950408889634


def _proj_body(f, n_mechs, x_ref, w_ref, asrc_ref, adst_ref,
               h_ref, ls_ref, ld_ref, hmean_ref):
    h = jnp.dot(x_ref[...], w_ref[...], preferred_element_type=jnp.float32)
    ls = jnp.dot(h, asrc_ref[...], preferred_element_type=jnp.float32)
    ld = jnp.dot(h, adst_ref[...], preferred_element_type=jnp.float32)
    b = ls[:, :4] + jnp.max(ld[:, :4], axis=0, keepdims=True)
    mp = jnp.maximum(b, _LEAK * b)
    c = jnp.max(mp, axis=0, keepdims=True)
    cvec = jnp.concatenate([c, c], axis=1)
    ls_ref[...] = ls.astype(jnp.bfloat16)
    ld_ref[...] = (ld - cvec).astype(jnp.bfloat16)
    hmean_ref[...] = jnp.broadcast_to(
        jnp.mean(h, axis=0, keepdims=True), hmean_ref.shape)
    stride = f + 8
    hb = h.astype(jnp.bfloat16)
    for m in range(n_mechs):
        h_ref[:, m * stride : m * stride + f] = hb[:, m * f : (m + 1) * f]
        h_ref[:, m * stride + f : m * stride + stride] = jnp.ones(
            (h.shape[0], 8), jnp.bfloat16)


def _project(x, w, asrc, adst, n_mechs, f):
    n = x.shape[0]
    body = lambda *refs: _proj_body(f, n_mechs, *refs)
    return pl.pallas_call(
        body,
        out_shape=[
            jax.ShapeDtypeStruct((n, n_mechs * (f + 8)), jnp.bfloat16),
            jax.ShapeDtypeStruct((n, 8), jnp.bfloat16),
            jax.ShapeDtypeStruct((n, 8), jnp.bfloat16),
            jax.ShapeDtypeStruct((8, n_mechs * f), jnp.float32),
        ],
    )(x, w, asrc, adst)


def _mech_loop(n_mechs, f, adjb, ls_ref, ldt_ref, h_ref, hmean_ref, out_ref):
    stride = f + 8
    for m in range(n_mechs):
        u = ls_ref[:, m : m + 1] + ldt_ref[m : m + 1, :]
        v = ls_ref[:, 4 + m : 5 + m] + ldt_ref[4 + m : 5 + m, :]
        p = jnp.exp2(jnp.maximum(u, v)) * adjb
        acc = jnp.dot(p, h_ref[:, m * stride : (m + 1) * stride],
                      preferred_element_type=jnp.float32)
        den = acc[:, f : f + 1]
        o = jnp.where(den > 0.0, acc[:, :f] / den,
                      hmean_ref[0:1, m * f : (m + 1) * f])
        out_ref[:, m * f : (m + 1) * f] = jnp.where(
            o > 0.0, o, jnp.exp(o) - 1.0)


def _attn1_body(n_mechs, f, adj_ref, ls_ref, ldt_ref, h_ref,
                hmean_ref, out_ref, adjb_ref):
    adjb = adj_ref[...].astype(jnp.bfloat16)
    adjb_ref[...] = adjb
    _mech_loop(n_mechs, f, adjb, ls_ref, ldt_ref, h_ref, hmean_ref, out_ref)


def _attn2_body(n_mechs, f, adjb_ref, ls_ref, ldt_ref, h_ref,
                hmean_ref, out_ref):
    _mech_loop(n_mechs, f, adjb_ref[...], ls_ref, ldt_ref, h_ref,
               hmean_ref, out_ref)


def _attention(adj, ls, ldt, h, hmean, n_mechs, f, block_rows, emit_adjb):
    n = adj.shape[0]
    row_specs = [
        pl.BlockSpec((block_rows, n), lambda i: (i, 0)),
        pl.BlockSpec((block_rows, 8), lambda i: (i, 0)),
        pl.BlockSpec((8, n), lambda i: (0, 0)),
        pl.BlockSpec((n, n_mechs * (f + 8)), lambda i: (0, 0)),
        pl.BlockSpec((8, n_mechs * f), lambda i: (0, 0)),
    ]
    out_spec = pl.BlockSpec((block_rows, n_mechs * f), lambda i: (i, 0))
    if emit_adjb:
        body = lambda *refs: _attn1_body(n_mechs, f, *refs)
        return pl.pallas_call(
            body,
            grid=(n // block_rows,),
            in_specs=row_specs,
            out_specs=[out_spec,
                       pl.BlockSpec((block_rows, n), lambda i: (i, 0))],
            out_shape=[
                jax.ShapeDtypeStruct((n, n_mechs * f), jnp.float32),
                jax.ShapeDtypeStruct((n, n), jnp.bfloat16),
            ],
        )(adj, ls, ldt, h, hmean)
    body = lambda *refs: _attn2_body(n_mechs, f, *refs)
    return pl.pallas_call(
        body,
        grid=(n // block_rows,),
        in_specs=row_specs,
        out_specs=out_spec,
        out_shape=jax.ShapeDtypeStruct((n, n_mechs * f), jnp.float32),
    )(adj, ls, ldt, h, hmean)


def kernel(x, adj, Ws, As, W_out, a_out):
    n, ins = x.shape
    n_mechs, _, f1 = Ws.shape
    out_dim = W_out.shape[1]
    block_rows = 200 if n % 200 == 0 else (8 if n % 8 == 0 else 1)

    w_cat = jnp.transpose(Ws, (1, 0, 2)).reshape(ins, n_mechs * f1)
    asrc = jnp.zeros((n_mechs * f1, 8), jnp.float32)
    adst = jnp.zeros((n_mechs * f1, 8), jnp.float32)
    for m in range(n_mechs):
        asrc = asrc.at[m * f1 : (m + 1) * f1, m].set(As[m, :f1] * _LOG2E)
        adst = adst.at[m * f1 : (m + 1) * f1, m].set(As[m, f1:] * _LOG2E)
    asrc = asrc.at[:, 4:].set(asrc[:, :4] * _LEAK)
    adst = adst.at[:, 4:].set(adst[:, :4] * _LEAK)

    h1, ls1, ld1, hm1 = _project(x, w_cat, asrc, adst, n_mechs, f1)
    xc, adjb = _attention(adj, ls1, ld1.T, h1, hm1, n_mechs, f1,
                          block_rows, True)

    a2s = jnp.zeros((out_dim, 8), jnp.float32).at[:, 0].set(
        a_out[:out_dim] * _LOG2E)
    a2d = jnp.zeros((out_dim, 8), jnp.float32).at[:, 0].set(
        a_out[out_dim:] * _LOG2E)
    a2s = a2s.at[:, 4:].set(a2s[:, :4] * _LEAK)
    a2d = a2d.at[:, 4:].set(a2d[:, :4] * _LEAK)
    h2, ls2, ld2, hm2 = _project(xc, W_out, a2s, a2d, 1, out_dim)
    out = _attention(adjb, ls2, ld2.T, h2, hm2, 1, out_dim,
                     block_rows, False)
    return out


# speedup vs baseline: 6.5550x; 1.2114x over previous
import jax
import jax.numpy as jnp
from jax.experimental import pallas as pl

_LEAK = 0.2
_LOG2E = 1.4426950408889634


def _proj_body(f, n_mechs, x_ref, w_ref, asrc_ref, adst_ref,
               h_ref, ls_ref, ld_ref, hmean_ref):
    h = jnp.dot(x_ref[...], w_ref[...], preferred_element_type=jnp.float32)
    ls = jnp.dot(h, asrc_ref[...], preferred_element_type=jnp.float32)
    ld = jnp.dot(h, adst_ref[...], preferred_element_type=jnp.float32)
    b = ls[:, :4] + jnp.max(ld[:, :4], axis=0, keepdims=True)
    mp = jnp.maximum(b, _LEAK * b)
    c = jnp.max(mp, axis=0, keepdims=True)
    cvec = jnp.concatenate([c, c], axis=1)
    ls_ref[...] = ls.astype(jnp.bfloat16)
    ld_ref[...] = (ld - cvec).astype(jnp.bfloat16)
    hmean_ref[...] = jnp.broadcast_to(
        jnp.mean(h, axis=0, keepdims=True), hmean_ref.shape)
    stride = f + 8
    hb = h.astype(jnp.bfloat16)
    for m in range(n_mechs):
        h_ref[:, m * stride : m * stride + f] = hb[:, m * f : (m + 1) * f]
        h_ref[:, m * stride + f : m * stride + stride] = jnp.ones(
            (h.shape[0], 8), jnp.bfloat16)


def _project(x, w, asrc, adst, n_mechs, f):
    n = x.shape[0]
    body = lambda *refs: _proj_body(f, n_mechs, *refs)
    return pl.pallas_call(
        body,
        out_shape=[
            jax.ShapeDtypeStruct((n, n_mechs * (f + 8)), jnp.bfloat16),
            jax.ShapeDtypeStruct((n, 8), jnp.bfloat16),
            jax.ShapeDtypeStruct((n, 8), jnp.bfloat16),
            jax.ShapeDtypeStruct((8, n_mechs * f), jnp.float32),
        ],
    )(x, w, asrc, adst)


def _mech_loop(n_mechs, f, adjb, ls_ref, ldt_ref, h_ref, hmean_ref, out_ref):
    stride = f + 8
    for m in range(n_mechs):
        u = ls_ref[:, m : m + 1] + ldt_ref[m : m + 1, :]
        v = ls_ref[:, 4 + m : 5 + m] + ldt_ref[4 + m : 5 + m, :]
        p = jnp.exp2(jnp.maximum(u, v)) * adjb
        acc = jnp.dot(p, h_ref[:, m * stride : (m + 1) * stride],
                      preferred_element_type=jnp.float32)
        den = acc[:, f : f + 1]
        o = jnp.where(den > 0.0, acc[:, :f] / den,
                      hmean_ref[0:1, m * f : (m + 1) * f])
        out_ref[:, m * f : (m + 1) * f] = jnp.where(
            o > 0.0, o, jnp.exp(o) - 1.0)


def _attn1_body(n_mechs, f, adj_ref, ls_ref, ldt_ref, h_ref,
                hmean_ref, out_ref, adjb_ref):
    adjb = adj_ref[...].astype(jnp.bfloat16)
    adjb_ref[...] = adjb
    _mech_loop(n_mechs, f, adjb, ls_ref, ldt_ref, h_ref, hmean_ref, out_ref)


def _attn2_body(n_mechs, f, adjb_ref, ls_ref, ldt_ref, h_ref,
                hmean_ref, out_ref):
    _mech_loop(n_mechs, f, adjb_ref[...], ls_ref, ldt_ref, h_ref,
               hmean_ref, out_ref)


def _attention(adj, ls, ldt, h, hmean, n_mechs, f, block_rows, emit_adjb):
    n = adj.shape[0]
    row_specs = [
        pl.BlockSpec((block_rows, n), lambda i: (i, 0)),
        pl.BlockSpec((block_rows, 8), lambda i: (i, 0)),
        pl.BlockSpec((8, n), lambda i: (0, 0)),
        pl.BlockSpec((n, n_mechs * (f + 8)), lambda i: (0, 0)),
        pl.BlockSpec((8, n_mechs * f), lambda i: (0, 0)),
    ]
    out_spec = pl.BlockSpec((block_rows, n_mechs * f), lambda i: (i, 0))
    if emit_adjb:
        body = lambda *refs: _attn1_body(n_mechs, f, *refs)
        return pl.pallas_call(
            body,
            grid=(n // block_rows,),
            in_specs=row_specs,
            out_specs=[out_spec,
                       pl.BlockSpec((block_rows, n), lambda i: (i, 0))],
            out_shape=[
                jax.ShapeDtypeStruct((n, n_mechs * f), jnp.float32),
                jax.ShapeDtypeStruct((n, n), jnp.bfloat16),
            ],
        )(adj, ls, ldt, h, hmean)
    body = lambda *refs: _attn2_body(n_mechs, f, *refs)
    return pl.pallas_call(
        body,
        grid=(n // block_rows,),
        in_specs=row_specs,
        out_specs=out_spec,
        out_shape=jax.ShapeDtypeStruct((n, n_mechs * f), jnp.float32),
    )(adj, ls, ldt, h, hmean)


def kernel(x, adj, Ws, As, W_out, a_out):
    n, ins = x.shape
    n_mechs, _, f1 = Ws.shape
    out_dim = W_out.shape[1]
    block_rows = 400 if n % 400 == 0 else (8 if n % 8 == 0 else 1)

    w_cat = jnp.transpose(Ws, (1, 0, 2)).reshape(ins, n_mechs * f1)
    asrc = jnp.zeros((n_mechs * f1, 8), jnp.float32)
    adst = jnp.zeros((n_mechs * f1, 8), jnp.float32)
    for m in range(n_mechs):
        asrc = asrc.at[m * f1 : (m + 1) * f1, m].set(As[m, :f1] * _LOG2E)
        adst = adst.at[m * f1 : (m + 1) * f1, m].set(As[m, f1:] * _LOG2E)
    asrc = asrc.at[:, 4:].set(asrc[:, :4] * _LEAK)
    adst = adst.at[:, 4:].set(adst[:, :4] * _LEAK)

    h1, ls1, ld1, hm1 = _project(x, w_cat, asrc, adst, n_mechs, f1)
    xc, adjb = _attention(adj, ls1, ld1.T, h1, hm1, n_mechs, f1,
                          block_rows, True)

    a2s = jnp.zeros((out_dim, 8), jnp.float32).at[:, 0].set(
        a_out[:out_dim] * _LOG2E)
    a2d = jnp.zeros((out_dim, 8), jnp.float32).at[:, 0].set(
        a_out[out_dim:] * _LOG2E)
    a2s = a2s.at[:, 4:].set(a2s[:, :4] * _LEAK)
    a2d = a2d.at[:, 4:].set(a2d[:, :4] * _LEAK)
    h2, ls2, ld2, hm2 = _project(xc, W_out, a2s, a2d, 1, out_dim)
    out = _attention(adjb, ls2, ld2.T, h2, hm2, 1, out_dim,
                     block_rows, False)
    return out


# speedup vs baseline: 6.6055x; 1.0077x over previous
import jax
import jax.numpy as jnp
from jax.experimental import pallas as pl
from jax.experimental.pallas import tpu as pltpu

_LEAK = 0.2
_LOG2E = 1.4426950408889634


def _proj_compute(n_mechs, f, x_ref, w_ref, asrc_ref, adst_ref,
                  h_out, ls_out, ldt_out, hm_out, transpose_ld):
    h = jnp.dot(x_ref[...], w_ref[...], preferred_element_type=jnp.float32)
    ls = jnp.dot(h, asrc_ref[...], preferred_element_type=jnp.float32)
    ld = jnp.dot(h, adst_ref[...], preferred_element_type=jnp.float32)
    b = ls[:, :4] + jnp.max(ld[:, :4], axis=0, keepdims=True)
    mp = jnp.maximum(b, _LEAK * b)
    c = jnp.max(mp, axis=0, keepdims=True)
    cvec = jnp.concatenate([c, c], axis=1)
    ls_out[...] = ls.astype(jnp.bfloat16)
    ldc = (ld - cvec).astype(jnp.bfloat16)
    ldt_out[...] = ldc.T if transpose_ld else ldc
    hm_out[...] = jnp.broadcast_to(
        jnp.mean(h, axis=0, keepdims=True), hm_out.shape)
    stride = f + 8
    hb = h.astype(jnp.bfloat16)
    for m in range(n_mechs):
        h_out[:, m * stride : m * stride + f] = hb[:, m * f : (m + 1) * f]
        h_out[:, m * stride + f : m * stride + stride] = jnp.ones(
            (h.shape[0], 8), jnp.bfloat16)


def _project(x, w, asrc, adst, n_mechs, f):
    n = x.shape[0]
    body = lambda x_ref, w_ref, s_ref, d_ref, h_ref, ls_ref, ld_ref, hm_ref: (
        _proj_compute(n_mechs, f, x_ref, w_ref, s_ref, d_ref,
                      h_ref, ls_ref, ld_ref, hm_ref, False))
    return pl.pallas_call(
        body,
        out_shape=[
            jax.ShapeDtypeStruct((n, n_mechs * (f + 8)), jnp.bfloat16),
            jax.ShapeDtypeStruct((n, 8), jnp.bfloat16),
            jax.ShapeDtypeStruct((n, 8), jnp.bfloat16),
            jax.ShapeDtypeStruct((8, n_mechs * f), jnp.float32),
        ],
    )(x, w, asrc, adst)


def _mech_loop(n_mechs, f, adjb, ls, ldt_ref, h_ref, hm_ref, out_ref):
    stride = f + 8
    for m in range(n_mechs):
        u = ls[:, m : m + 1] + ldt_ref[m : m + 1, :]
        v = ls[:, 4 + m : 5 + m] + ldt_ref[4 + m : 5 + m, :]
        p = jnp.exp2(jnp.maximum(u, v)) * adjb
        acc = jnp.dot(p, h_ref[:, m * stride : (m + 1) * stride],
                      preferred_element_type=jnp.float32)
        den = acc[:, f : f + 1]
        o = jnp.where(den > 0.0, acc[:, :f] / den,
                      hm_ref[0:1, m * f : (m + 1) * f])
        out_ref[:, m * f : (m + 1) * f] = jnp.where(
            o > 0.0, o, jnp.exp(o) - 1.0)


def _attn1_body(n_mechs, f, adj_ref, ls_ref, ldt_ref, h_ref,
                hm_ref, out_ref, adjb_ref):
    adjb = adj_ref[...].astype(jnp.bfloat16)
    adjb_ref[...] = adjb
    _mech_loop(n_mechs, f, adjb, ls_ref[...], ldt_ref, h_ref, hm_ref,
               out_ref)


def _attention1(adj, ls, ldt, h, hmean, n_mechs, f, block_rows):
    n = adj.shape[0]
    body = lambda *refs: _attn1_body(n_mechs, f, *refs)
    return pl.pallas_call(
        body,
        grid=(n // block_rows,),
        in_specs=[
            pl.BlockSpec((block_rows, n), lambda i: (i, 0)),
            pl.BlockSpec((block_rows, 8), lambda i: (i, 0)),
            pl.BlockSpec((8, n), lambda i: (0, 0)),
            pl.BlockSpec((n, n_mechs * (f + 8)), lambda i: (0, 0)),
            pl.BlockSpec((8, n_mechs * f), lambda i: (0, 0)),
        ],
        out_specs=[
            pl.BlockSpec((block_rows, n_mechs * f), lambda i: (i, 0)),
            pl.BlockSpec((block_rows, n), lambda i: (i, 0)),
        ],
        out_shape=[
            jax.ShapeDtypeStruct((n, n_mechs * f), jnp.float32),
            jax.ShapeDtypeStruct((n, n), jnp.bfloat16),
        ],
    )(adj, ls, ldt, h, hmean)


def _layer2_body(n_mechs, f, block_rows, x_ref, w_ref, asrc_ref, adst_ref,
                 adjb_ref, out_ref, h_scr, ls_scr, ldt_scr, hm_scr):
    i = pl.program_id(0)

    @pl.when(i == 0)
    def _():
        _proj_compute(n_mechs, f, x_ref, w_ref, asrc_ref, adst_ref,
                      h_scr, ls_scr, ldt_scr, hm_scr, True)

    @pl.when(i > 0)
    def _():
        r0 = (i - 1) * block_rows
        ls_blk = ls_scr[pl.ds(r0, block_rows), :]
        _mech_loop(n_mechs, f, adjb_ref[...], ls_blk, ldt_scr, h_scr,
                   hm_scr, out_ref)


def _layer2(xc, w, asrc, adst, adjb, n_mechs, f, block_rows):
    n = adjb.shape[0]
    clamp = lambda i: (jnp.maximum(i - 1, 0), 0)
    const = lambda i: (0, 0)
    body = lambda *refs: _layer2_body(n_mechs, f, block_rows, *refs)
    return pl.pallas_call(
        body,
        grid=(n // block_rows + 1,),
        in_specs=[
            pl.BlockSpec(xc.shape, const),
            pl.BlockSpec(w.shape, const),
            pl.BlockSpec((w.shape[1], 8), const),
            pl.BlockSpec((w.shape[1], 8), const),
            pl.BlockSpec((block_rows, n), clamp),
        ],
        out_specs=pl.BlockSpec((block_rows, n_mechs * f), clamp),
        out_shape=jax.ShapeDtypeStruct((n, n_mechs * f), jnp.float32),
        scratch_shapes=[
            pltpu.VMEM((n, n_mechs * (f + 8)), jnp.bfloat16),
            pltpu.VMEM((n, 8), jnp.bfloat16),
            pltpu.VMEM((8, n), jnp.bfloat16),
            pltpu.VMEM((8, n_mechs * f), jnp.float32),
        ],
    )(xc, w, asrc, adst, adjb)


def kernel(x, adj, Ws, As, W_out, a_out):
    n, ins = x.shape
    n_mechs, _, f1 = Ws.shape
    out_dim = W_out.shape[1]
    block_rows = 400 if n % 400 == 0 else (8 if n % 8 == 0 else 1)

    w_cat = jnp.transpose(Ws, (1, 0, 2)).reshape(ins, n_mechs * f1)
    asrc = jnp.zeros((n_mechs * f1, 8), jnp.float32)
    adst = jnp.zeros((n_mechs * f1, 8), jnp.float32)
    for m in range(n_mechs):
        asrc = asrc.at[m * f1 : (m + 1) * f1, m].set(As[m, :f1] * _LOG2E)
        adst = adst.at[m * f1 : (m + 1) * f1, m].set(As[m, f1:] * _LOG2E)
    asrc = asrc.at[:, 4:].set(asrc[:, :4] * _LEAK)
    adst = adst.at[:, 4:].set(adst[:, :4] * _LEAK)

    h1, ls1, ld1, hm1 = _project(x, w_cat, asrc, adst, n_mechs, f1)
    xc, adjb = _attention1(adj, ls1, ld1.T, h1, hm1, n_mechs, f1,
                           block_rows)

    a2s = jnp.zeros((out_dim, 8), jnp.float32).at[:, 0].set(
        a_out[:out_dim] * _LOG2E)
    a2d = jnp.zeros((out_dim, 8), jnp.float32).at[:, 0].set(
        a_out[out_dim:] * _LOG2E)
    a2s = a2s.at[:, 4:].set(a2s[:, :4] * _LEAK)
    a2d = a2d.at[:, 4:].set(a2d[:, :4] * _LEAK)
    out = _layer2(xc, W_out, a2s, a2d, adjb, 1, out_dim, block_rows)
    return out
